# baseline (speedup 1.0000x reference)
import functools

import jax
import jax.numpy as jnp
from jax import lax
from jax.experimental import pallas as pl
from jax.experimental.pallas import tpu as pltpu
from jax.experimental.pallas import tpu_sc as plsc

B = 4096
F = 26
D = 32
N_ATN = 16

_info = plsc.get_sparse_core_info()
_NC, _NS, _L = _info.num_cores, _info.num_subcores, _info.num_lanes
NW = _NC * _NS
BPW = B // NW

_mesh = plsc.VectorSubcoreMesh(core_axis_name="c", subcore_axis_name="s")


@functools.partial(
    pl.kernel,
    mesh=_mesh,
    out_type=jax.ShapeDtypeStruct((B, D), jnp.float32),
    scratch_types=[
        pltpu.VMEM((F, BPW), jnp.int32),
        pltpu.VMEM((F, BPW, D), jnp.float32),
        pltpu.VMEM((BPW, D), jnp.float32),
        pltpu.SemaphoreType.DMA,
    ],
    compiler_params=pltpu.CompilerParams(use_tc_tiling_on_sc=False),
)
def _sc_gather_sum(idx_hbm, table_hbm, out_hbm, idx_v, rows_v, out_v, sem):
    wid = lax.axis_index("s") * _NC + lax.axis_index("c")
    base = wid * BPW

    pltpu.sync_copy(idx_hbm.at[wid], idx_v)

    copies = [
        pltpu.async_copy(table_hbm.at[idx_v.at[f]], rows_v.at[f], sem)
        for f in range(F)
    ]
    for c in copies:
        c.wait()

    def body(b, carry):
        for col in (0, _L):
            a0 = rows_v[0, b, pl.ds(col, _L)] + rows_v[1, b, pl.ds(col, _L)]
            a1 = rows_v[2, b, pl.ds(col, _L)] + rows_v[3, b, pl.ds(col, _L)]
            for f in range(4, F, 2):
                a0 = a0 + rows_v[f, b, pl.ds(col, _L)]
                a1 = a1 + rows_v[f + 1, b, pl.ds(col, _L)]
            out_v[b, pl.ds(col, _L)] = a0 + a1
        return carry

    lax.fori_loop(0, BPW, body, 0)

    pltpu.sync_copy(out_v, out_hbm.at[pl.ds(base, BPW)])


def _combine_body(sum_ref, val_ref, atn_ref, w_ref, b_ref, act_ref, out_ref):
    inv2f = 1.0 / (2.0 * F)
    s = jnp.sum(val_ref[...], axis=1, keepdims=True)
    onehot = (atn_ref[...] ==
              lax.broadcasted_iota(jnp.int32, (B, N_ATN), 1)
              ).astype(jnp.float32)
    act = jnp.dot(onehot, act_ref[...],
                  preferred_element_type=jnp.float32)
    out_ref[...] = (sum_ref[...] * inv2f
                    + s * (w_ref[...] * inv2f)
                    + b_ref[...] * 0.5
                    + act)


def kernel(idx_d, val_c, atn_idx, table_d, W_c, b_c, action_table):
    idx_r = (idx_d.astype(jnp.int32).T
             .reshape(F, NW, BPW)
             .transpose(1, 0, 2))

    sums = _sc_gather_sum(idx_r, table_d)

    out = pl.pallas_call(
        _combine_body,
        out_shape=jax.ShapeDtypeStruct((B, D), jnp.float32),
    )(
        sums,
        val_c,
        atn_idx.astype(jnp.int32).reshape(B, 1),
        W_c.astype(jnp.float32).reshape(1, D),
        b_c.reshape(1, D),
        action_table,
    )
    return out
